# baseline (speedup 1.0000x reference)
import functools

import jax
import jax.numpy as jnp
from jax import lax
from jax.experimental import pallas as pl
from jax.experimental.pallas import tpu as pltpu
from jax.experimental.pallas import tpu_sc as plsc

_NROWS = 26
_W = 16384
_TOTAL_TR = _NROWS * 4
_NW = 32
_KMAX = 4

_mesh = plsc.VectorSubcoreMesh(core_axis_name="c", subcore_axis_name="s")


@functools.partial(
    pl.kernel,
    out_type=jax.ShapeDtypeStruct((_TOTAL_TR * 8, _W), jnp.float32),
    mesh=_mesh,
    scratch_types=[
        pltpu.VMEM((16,), jnp.int32),
        pltpu.SemaphoreType.DMA,
    ],
)
def _sc_gather(table, idx_hbm, out, idxv, sem):
    vid = lax.axis_index("s") * 2 + lax.axis_index("c")
    pltpu.sync_copy(idx_hbm.at[vid], idxv)
    srows = idxv[...]
    for k in range(_KMAX):
        u = vid + _NW * k

        @pl.when(u < _TOTAL_TR)
        def _():
            s = srows[k]
            pltpu.make_async_copy(
                table.at[pl.ds(s * 8, 8), :], out.at[pl.ds(u * 8, 8), :], sem
            ).start()
    for k in range(_KMAX):
        u = vid + _NW * k

        @pl.when(u < _TOTAL_TR)
        def _():
            s = srows[k]
            pltpu.make_async_copy(
                table.at[pl.ds(s * 8, 8), :], out.at[pl.ds(u * 8, 8), :], sem
            ).wait()


def kernel(mamdani_output, mapping):
    src = jnp.transpose(mamdani_output, (0, 2, 1)).reshape(3200, _W)
    v = jnp.arange(32, dtype=jnp.int32)[:, None]
    k = jnp.arange(16, dtype=jnp.int32)[None, :]
    u = jnp.minimum(v + _NW * k, _TOTAL_TR - 1)
    idx = mapping.reshape(_NROWS)[u // 4] * 4 + (u % 4)
    out = _sc_gather(src, idx)
    out = jnp.transpose(out.reshape(_NROWS, 32, _W), (0, 2, 1))
    return jnp.expand_dims(out, 1)


# speedup vs baseline: 29.5553x; 29.5553x over previous
import functools

import jax
import jax.numpy as jnp
from jax import lax
from jax.experimental import pallas as pl
from jax.experimental.pallas import tpu as pltpu
from jax.experimental.pallas import tpu_sc as plsc

_NROWS = 26
_W = 16384
_H = _W // 2
_TOTAL_U = _NROWS * 8
_NSLOT = 15
_NACT = 2 * _NSLOT
_KMAX = 7

_mesh = plsc.VectorSubcoreMesh(core_axis_name="c", subcore_axis_name="s")


@functools.partial(
    pl.kernel,
    out_type=jax.ShapeDtypeStruct((_NROWS * 32, _W), jnp.float32),
    mesh=_mesh,
    scratch_types=[
        pltpu.VMEM((16,), jnp.int32),
        pltpu.VMEM_SHARED((_NSLOT, 2, 8, _H), jnp.float32),
        pltpu.SemaphoreType.DMA((2,)),
        pltpu.SemaphoreType.DMA((2,)),
    ],
)
def _sc_gather(table, idx_hbm, out, idxv, shared, gsem, psem):
    sid = lax.axis_index("s")
    cid = lax.axis_index("c")
    vid = cid * _NSLOT + sid
    active = sid < _NSLOT

    @pl.when(active)
    def _():
        pltpu.sync_copy(idx_hbm.at[vid], idxv)

    srows = idxv[...]

    def unit(j):
        return j * _NACT + vid

    def valid(j):
        return active & (unit(j) < _TOTAL_U)

    def gather(j):
        u = unit(j)
        s = srows[j]
        return pltpu.make_async_copy(
            table.at[pl.ds(s * 8, 8), pl.ds((u % 2) * _H, _H)],
            shared.at[sid, j % 2],
            gsem.at[j % 2],
        )

    def put(j):
        u = unit(j)
        return pltpu.make_async_copy(
            shared.at[sid, j % 2],
            out.at[pl.ds((u // 2) * 8, 8), pl.ds((u % 2) * _H, _H)],
            psem.at[j % 2],
        )

    @pl.when(valid(0))
    def _():
        gather(0).start()

    for j in range(_KMAX):

        @pl.when(valid(j))
        def _(j=j):
            gather(j).wait()
            put(j).start()

        if j + 1 < _KMAX:

            @pl.when(valid(j + 1))
            def _(j=j):
                if j >= 1:
                    put(j - 1).wait()
                gather(j + 1).start()

    @pl.when(valid(_KMAX - 1))
    def _():
        put(_KMAX - 1).wait()
    for j in (_KMAX - 2, _KMAX - 3):

        @pl.when(valid(j) & ~valid(j + 2))
        def _(j=j):
            put(j).wait()


def kernel(mamdani_output, mapping):
    src = jnp.transpose(mamdani_output, (0, 2, 1)).reshape(3200, _W)
    m = mapping.reshape(_NROWS)
    j = jnp.arange(16, dtype=jnp.int32)[None, :]
    v = jnp.arange(32, dtype=jnp.int32)[:, None]
    u = jnp.minimum(j * _NACT + v, _TOTAL_U - 1)
    tr = u // 2
    idx = m[tr // 4] * 4 + (tr % 4)
    out = _sc_gather(src, idx)
    out = jnp.transpose(out.reshape(_NROWS, 32, _W), (0, 2, 1))
    return jnp.expand_dims(out, 1)
